# baseline (speedup 1.0000x reference)
import functools

import jax
import jax.numpy as jnp
from jax import lax
from jax.experimental import pallas as pl
from jax.experimental.pallas import tpu as pltpu
from jax.experimental.pallas import tpu_sc as plsc

N = 50000
E = 500000
NP = 50176
STRIPE = NP // 16
CHUNK = 512
K_CHUNKS = 63
PT = CHUNK * K_CHUNKS
G = 7
EP = 16 * PT
NG = K_CHUNKS // G
NB = 1

ETYPES = ['uv', 'up', 'vu', 'vt', 'pu', 'pt', 'tv', 'tp']
SIG = [2, 4, 0, 6, 1, 7, 3, 5]
NT_ETYPES = [[2, 4], [0, 6], [1, 7], [3, 5]]

_MESH = plsc.VectorSubcoreMesh(core_axis_name="c", subcore_axis_name="s")
_SC_PARAMS = pltpu.CompilerParams(use_tc_tiling_on_sc=False)


@functools.partial(
    pl.kernel,
    out_type=jax.ShapeDtypeStruct((8, NP, 32), jnp.float32),
    mesh=_MESH,
    compiler_params=_SC_PARAMS,
    scratch_types=[
        pltpu.VMEM_SHARED((NP, 32), jnp.float32),
        pltpu.VMEM((G, CHUNK), jnp.int32),
        pltpu.VMEM((CHUNK, 32), jnp.float32),
    ],
)
def _sc_counts(dst3, z32, ones32, cnt_out, acc, stage, ones_v):
    c = lax.axis_index("c")
    s = lax.axis_index("s")
    pltpu.sync_copy(ones32, ones_v)
    for half in range(2):
        @pl.when(c == half)
        def _():
            for i in range(4):
                e = 4 * half + i
                pltpu.sync_copy(z32, acc.at[pl.ds(s * STRIPE, STRIPE)])
                plsc.subcore_barrier()

                def group(g, carry):
                    pltpu.sync_copy(dst3.at[e, s, g], stage)

                    def chunk(j, carry2):
                        pltpu.sync_copy(ones_v, acc.at[stage.at[j]], add=True)
                        return carry2

                    lax.fori_loop(0, G, chunk, 0)
                    return carry

                lax.fori_loop(0, NG, group, 0)
                plsc.subcore_barrier()
                pltpu.sync_copy(acc.at[pl.ds(s * STRIPE, STRIPE)],
                                cnt_out.at[SIG[e], pl.ds(s * STRIPE, STRIPE)])
                plsc.subcore_barrier()


def _make_sc_sums(permute_out):
    @functools.partial(
        pl.kernel,
        out_type=jax.ShapeDtypeStruct((16, NP, 32), jnp.float32),
        mesh=_MESH,
        compiler_params=_SC_PARAMS,
        scratch_types=[
            pltpu.VMEM_SHARED((NP, 32), jnp.float32),
            pltpu.VMEM((G, CHUNK), jnp.int32),
            pltpu.VMEM((G, CHUNK), jnp.int32),
        ] + [pltpu.VMEM((CHUNK, 32), jnp.float32) for _ in range(NB)]
          + [pltpu.SemaphoreType.DMA for _ in range(NB)],
    )
    def _sums(tables, src3, dst3, z32, sums_out, acc, sstage, dstage, *rs):
        rows = rs[:NB]
        sems = rs[NB:]
        c = lax.axis_index("c")
        s = lax.axis_index("s")
        for e in range(8):
            k_in = 2 * e + c
            k_out = 2 * (SIG[e] if permute_out else e) + c
            pltpu.sync_copy(z32, acc.at[pl.ds(s * STRIPE, STRIPE)])
            plsc.subcore_barrier()
            tab = tables.at[k_in]

            def group(g, carry):
                pltpu.sync_copy(src3.at[e, s, g], sstage)
                pltpu.sync_copy(dst3.at[e, s, g], dstage)
                hand = [pltpu.async_copy(tab.at[sstage.at[b]], rows[b], sems[b])
                        for b in range(NB)]
                for jj in range(G):
                    b = jj % NB
                    hand[b].wait()
                    pltpu.sync_copy(rows[b], acc.at[dstage.at[jj]], add=True)
                    if jj + NB < G:
                        hand[b] = pltpu.async_copy(
                            tab.at[sstage.at[jj + NB]], rows[b], sems[b])
                return carry

            lax.fori_loop(0, NG, group, 0)
            plsc.subcore_barrier()
            pltpu.sync_copy(acc.at[pl.ds(s * STRIPE, STRIPE)],
                            sums_out.at[k_out, pl.ds(s * STRIPE, STRIPE)])
            plsc.subcore_barrier()

    return _sums


_sc_sums_l1 = _make_sc_sums(True)
_sc_sums_l2 = _make_sc_sums(False)


def _d1_body(s1_ref, c_ref, o_ref):
    inv = 1.0 / jnp.maximum(c_ref[...], 1.0)
    o_ref[...] = s1_ref[...] * inv


def _tc_divide(s1pr, cntpr):
    rows = NP // 4
    r1 = 1568
    nb = rows // r1
    return pl.pallas_call(
        _d1_body,
        grid=(16, nb),
        in_specs=[
            pl.BlockSpec((1, r1, 128), lambda k, b: (k, b, 0)),
            pl.BlockSpec((1, r1, 128), lambda k, b: (k // 2, b, 0)),
        ],
        out_specs=pl.BlockSpec((1, r1, 128), lambda k, b: (k, b, 0)),
        out_shape=jax.ShapeDtypeStruct((16, rows, 128), jnp.float32),
    )(s1pr, cntpr)


def _final_body(u_ref, v_ref, p_ref, t_ref, t2_ref, s2_ref, c_ref,
                ou_ref, ov_ref, op_ref, ot_ref):
    embs = [u_ref, v_ref, p_ref, t_ref]
    outs = [ou_ref, ov_ref, op_ref, ot_ref]
    third = 1.0 / 3.0
    for nt in range(4):
        for q in range(4):
            e_p = NT_ETYPES[nt][q // 2]
            cq = q % 2
            m1 = t2_ref[2 * SIG[e_p] + cq]
            inv = 1.0 / jnp.maximum(c_ref[SIG[e_p]], 1.0)
            m2 = s2_ref[2 * e_p + cq] * inv
            base = embs[nt][:, 32 * q:32 * (q + 1)]
            outs[nt][:, 32 * q:32 * (q + 1)] = base + 0.5 * m1 + third * m2


def _tc_final(u0, v0, p0, t0, t2_slab, s2_slab, cnt_slab):
    r = 1000
    nb = N // r
    emb_spec = pl.BlockSpec((r, 128), lambda b: (b, 0))
    return pl.pallas_call(
        _final_body,
        grid=(nb,),
        in_specs=[
            emb_spec, emb_spec, emb_spec, emb_spec,
            pl.BlockSpec((16, r, 32), lambda b: (0, b, 0)),
            pl.BlockSpec((16, r, 32), lambda b: (0, b, 0)),
            pl.BlockSpec((8, r, 32), lambda b: (0, b, 0)),
        ],
        out_specs=[emb_spec, emb_spec, emb_spec, emb_spec],
        out_shape=[jax.ShapeDtypeStruct((N, 128), jnp.float32)] * 4,
    )(u0, v0, p0, t0, t2_slab, s2_slab, cnt_slab)


def _build_tables(u, v, p, t):
    slabs = []
    for emb in (u, v, p, t):
        padded = jnp.pad(emb, ((0, NP - N), (0, 0)))
        slabs.append(padded.reshape(NP, 4, 32).transpose(1, 0, 2))
    return jnp.concatenate(slabs, axis=0)


def kernel(user_emb, video_emb, publisher_emb, tag_emb,
           edge_index_uv, edge_index_up, edge_index_vu, edge_index_vt,
           edge_index_pu, edge_index_pt, edge_index_tv, edge_index_tp):
    eis = [edge_index_uv, edge_index_up, edge_index_vu, edge_index_vt,
           edge_index_pu, edge_index_pt, edge_index_tv, edge_index_tp]
    srcs = jnp.stack([ei[0] for ei in eis])
    dsts = jnp.stack([ei[1] for ei in eis])
    src3 = jnp.pad(srcs, ((0, 0), (0, EP - E))).reshape(8, 16, NG, G, CHUNK)
    dst3 = jnp.pad(dsts, ((0, 0), (0, EP - E)),
                   constant_values=N).reshape(8, 16, NG, G, CHUNK)

    z32 = jnp.zeros((STRIPE, 32), jnp.float32)
    ones32 = jnp.ones((CHUNK, 32), jnp.float32)

    tables0 = _build_tables(user_emb, video_emb, publisher_emb, tag_emb)

    cntp = _sc_counts(dst3, z32, ones32)
    s1p = _sc_sums_l1(tables0, src3, dst3, z32)

    t2r = _tc_divide(s1p.reshape(16, NP // 4, 128),
                     cntp.reshape(8, NP // 4, 128))
    t2 = t2r.reshape(16, NP, 32)

    s2 = _sc_sums_l2(t2, src3, dst3, z32)

    return _tc_final(user_emb, video_emb, publisher_emb, tag_emb, t2, s2, cntp)


# speedup vs baseline: 1.0533x; 1.0533x over previous
import functools

import jax
import jax.numpy as jnp
from jax import lax
from jax.experimental import pallas as pl
from jax.experimental.pallas import tpu as pltpu
from jax.experimental.pallas import tpu_sc as plsc

N = 50000
E = 500000
NP = 50176
STRIPE = NP // 16
CHUNK = 128
K_CHUNKS = 245
PT = CHUNK * K_CHUNKS
G = 49
EP = 16 * PT
NG = K_CHUNKS // G
NB = 1

ETYPES = ['uv', 'up', 'vu', 'vt', 'pu', 'pt', 'tv', 'tp']
SIG = [2, 4, 0, 6, 1, 7, 3, 5]
NT_ETYPES = [[2, 4], [0, 6], [1, 7], [3, 5]]

_MESH = plsc.VectorSubcoreMesh(core_axis_name="c", subcore_axis_name="s")
_SC_PARAMS = pltpu.CompilerParams(use_tc_tiling_on_sc=False)


@functools.partial(
    pl.kernel,
    out_type=jax.ShapeDtypeStruct((8, NP, 32), jnp.float32),
    mesh=_MESH,
    compiler_params=_SC_PARAMS,
    scratch_types=[
        pltpu.VMEM_SHARED((NP, 32), jnp.float32),
        pltpu.VMEM((G, CHUNK), jnp.int32),
        pltpu.VMEM((CHUNK, 32), jnp.float32),
    ],
)
def _sc_counts(dst3, z32, ones32, cnt_out, acc, stage, ones_v):
    c = lax.axis_index("c")
    s = lax.axis_index("s")
    pltpu.sync_copy(ones32, ones_v)
    for half in range(2):
        @pl.when(c == half)
        def _():
            for i in range(4):
                e = 4 * half + i
                pltpu.sync_copy(z32, acc.at[pl.ds(s * STRIPE, STRIPE)])
                plsc.subcore_barrier()

                def group(g, carry):
                    pltpu.sync_copy(dst3.at[e, s, g], stage)

                    def chunk(j, carry2):
                        pltpu.sync_copy(ones_v, acc.at[stage.at[j]], add=True)
                        return carry2

                    lax.fori_loop(0, G, chunk, 0)
                    return carry

                lax.fori_loop(0, NG, group, 0)
                plsc.subcore_barrier()
                pltpu.sync_copy(acc.at[pl.ds(s * STRIPE, STRIPE)],
                                cnt_out.at[SIG[e], pl.ds(s * STRIPE, STRIPE)])
                plsc.subcore_barrier()


def _make_sc_sums(permute_out):
    @functools.partial(
        pl.kernel,
        out_type=jax.ShapeDtypeStruct((16, NP, 32), jnp.float32),
        mesh=_MESH,
        compiler_params=_SC_PARAMS,
        scratch_types=[
            pltpu.VMEM_SHARED((NP, 32), jnp.float32),
            pltpu.VMEM((G, CHUNK), jnp.int32),
            pltpu.VMEM((G, CHUNK), jnp.int32),
        ] + [pltpu.VMEM((CHUNK, 32), jnp.float32) for _ in range(NB)]
          + [pltpu.SemaphoreType.DMA for _ in range(NB)],
    )
    def _sums(tables, src3, dst3, z32, sums_out, acc, sstage, dstage, *rs):
        rows = rs[:NB]
        sems = rs[NB:]
        c = lax.axis_index("c")
        s = lax.axis_index("s")
        for e in range(8):
            k_in = 2 * e + c
            k_out = 2 * (SIG[e] if permute_out else e) + c
            pltpu.sync_copy(z32, acc.at[pl.ds(s * STRIPE, STRIPE)])
            plsc.subcore_barrier()
            tab = tables.at[k_in]

            def group(g, carry):
                pltpu.sync_copy(src3.at[e, s, g], sstage)
                pltpu.sync_copy(dst3.at[e, s, g], dstage)
                hand = [pltpu.async_copy(tab.at[sstage.at[b]], rows[b], sems[b])
                        for b in range(NB)]
                for jj in range(G):
                    b = jj % NB
                    hand[b].wait()
                    pltpu.sync_copy(rows[b], acc.at[dstage.at[jj]], add=True)
                    if jj + NB < G:
                        hand[b] = pltpu.async_copy(
                            tab.at[sstage.at[jj + NB]], rows[b], sems[b])
                return carry

            lax.fori_loop(0, NG, group, 0)
            plsc.subcore_barrier()
            pltpu.sync_copy(acc.at[pl.ds(s * STRIPE, STRIPE)],
                            sums_out.at[k_out, pl.ds(s * STRIPE, STRIPE)])
            plsc.subcore_barrier()

    return _sums


_sc_sums_l1 = _make_sc_sums(True)
_sc_sums_l2 = _make_sc_sums(False)


def _d1_body(s1_ref, c_ref, o_ref):
    inv = 1.0 / jnp.maximum(c_ref[...], 1.0)
    o_ref[...] = s1_ref[...] * inv


def _tc_divide(s1p, cntp):
    r1 = 3136
    nb = NP // r1
    return pl.pallas_call(
        _d1_body,
        grid=(16, nb),
        in_specs=[
            pl.BlockSpec((1, r1, 32), lambda k, b: (k, b, 0)),
            pl.BlockSpec((1, r1, 32), lambda k, b: (k // 2, b, 0)),
        ],
        out_specs=pl.BlockSpec((1, r1, 32), lambda k, b: (k, b, 0)),
        out_shape=jax.ShapeDtypeStruct((16, NP, 32), jnp.float32),
    )(s1p, cntp)


def _final_body(u_ref, v_ref, p_ref, t_ref, t2_ref, s2_ref, c_ref,
                ou_ref, ov_ref, op_ref, ot_ref):
    embs = [u_ref, v_ref, p_ref, t_ref]
    outs = [ou_ref, ov_ref, op_ref, ot_ref]
    third = 1.0 / 3.0
    for nt in range(4):
        for q in range(4):
            e_p = NT_ETYPES[nt][q // 2]
            cq = q % 2
            m1 = t2_ref[2 * SIG[e_p] + cq]
            inv = 1.0 / jnp.maximum(c_ref[SIG[e_p]], 1.0)
            m2 = s2_ref[2 * e_p + cq] * inv
            base = embs[nt][:, 32 * q:32 * (q + 1)]
            outs[nt][:, 32 * q:32 * (q + 1)] = base + 0.5 * m1 + third * m2


def _tc_final(u0, v0, p0, t0, t2_slab, s2_slab, cnt_slab):
    r = 1000
    nb = N // r
    emb_spec = pl.BlockSpec((r, 128), lambda b: (b, 0))
    return pl.pallas_call(
        _final_body,
        grid=(nb,),
        in_specs=[
            emb_spec, emb_spec, emb_spec, emb_spec,
            pl.BlockSpec((16, r, 32), lambda b: (0, b, 0)),
            pl.BlockSpec((16, r, 32), lambda b: (0, b, 0)),
            pl.BlockSpec((8, r, 32), lambda b: (0, b, 0)),
        ],
        out_specs=[emb_spec, emb_spec, emb_spec, emb_spec],
        out_shape=[jax.ShapeDtypeStruct((N, 128), jnp.float32)] * 4,
    )(u0, v0, p0, t0, t2_slab, s2_slab, cnt_slab)


def _build_tables(u, v, p, t):
    slabs = []
    for emb in (u, v, p, t):
        padded = jnp.pad(emb, ((0, NP - N), (0, 0)))
        slabs.append(padded.reshape(NP, 4, 32).transpose(1, 0, 2))
    return jnp.concatenate(slabs, axis=0)


def kernel(user_emb, video_emb, publisher_emb, tag_emb,
           edge_index_uv, edge_index_up, edge_index_vu, edge_index_vt,
           edge_index_pu, edge_index_pt, edge_index_tv, edge_index_tp):
    eis = [edge_index_uv, edge_index_up, edge_index_vu, edge_index_vt,
           edge_index_pu, edge_index_pt, edge_index_tv, edge_index_tp]
    srcs = jnp.stack([ei[0] for ei in eis])
    dsts = jnp.stack([ei[1] for ei in eis])
    src3 = jnp.pad(srcs, ((0, 0), (0, EP - E))).reshape(8, 16, NG, G, CHUNK)
    dst3 = jnp.pad(dsts, ((0, 0), (0, EP - E)),
                   constant_values=N).reshape(8, 16, NG, G, CHUNK)

    z32 = jnp.zeros((STRIPE, 32), jnp.float32)
    ones32 = jnp.ones((CHUNK, 32), jnp.float32)

    tables0 = _build_tables(user_emb, video_emb, publisher_emb, tag_emb)

    cntp = _sc_counts(dst3, z32, ones32)
    s1p = _sc_sums_l1(tables0, src3, dst3, z32)

    t2 = _tc_divide(s1p, cntp)

    s2 = _sc_sums_l2(t2, src3, dst3, z32)

    return _tc_final(user_emb, video_emb, publisher_emb, tag_emb, t2, s2, cntp)


# speedup vs baseline: 1.0558x; 1.0024x over previous
import functools

import jax
import jax.numpy as jnp
from jax import lax
from jax.experimental import pallas as pl
from jax.experimental.pallas import tpu as pltpu
from jax.experimental.pallas import tpu_sc as plsc

N = 50000
E = 500000
NP = 50176
STRIPE = NP // 16
CHUNK = 128
K_CHUNKS = 245
PT = CHUNK * K_CHUNKS
G = 49
EP = 16 * PT
NG = K_CHUNKS // G
NB = 1

ETYPES = ['uv', 'up', 'vu', 'vt', 'pu', 'pt', 'tv', 'tp']
SIG = [2, 4, 0, 6, 1, 7, 3, 5]
NT_ETYPES = [[2, 4], [0, 6], [1, 7], [3, 5]]

_MESH = plsc.VectorSubcoreMesh(core_axis_name="c", subcore_axis_name="s")
_SC_PARAMS = pltpu.CompilerParams(use_tc_tiling_on_sc=False)


@functools.partial(
    pl.kernel,
    out_type=jax.ShapeDtypeStruct((8, NP, 32), jnp.float32),
    mesh=_MESH,
    compiler_params=_SC_PARAMS,
    scratch_types=[
        pltpu.VMEM_SHARED((NP, 32), jnp.float32),
        pltpu.VMEM((G, CHUNK), jnp.int32),
        pltpu.VMEM((CHUNK, 32), jnp.float32),
    ],
)
def _sc_counts(dst3, z32, ones32, cnt_out, acc, stage, ones_v):
    c = lax.axis_index("c")
    s = lax.axis_index("s")
    pltpu.sync_copy(ones32, ones_v)
    for half in range(2):
        @pl.when(c == half)
        def _():
            for i in range(4):
                e = 4 * half + i
                pltpu.sync_copy(z32, acc.at[pl.ds(s * STRIPE, STRIPE)])
                plsc.subcore_barrier()

                def group(g, carry):
                    pltpu.sync_copy(dst3.at[e, s, g], stage)

                    def chunk(j, carry2):
                        pltpu.sync_copy(ones_v, acc.at[stage.at[j]], add=True)
                        return carry2

                    lax.fori_loop(0, G, chunk, 0)
                    return carry

                lax.fori_loop(0, NG, group, 0)
                plsc.subcore_barrier()
                pltpu.sync_copy(acc.at[pl.ds(s * STRIPE, STRIPE)],
                                cnt_out.at[SIG[e], pl.ds(s * STRIPE, STRIPE)])
                plsc.subcore_barrier()


def _make_sc_sums(permute_out):
    @functools.partial(
        pl.kernel,
        out_type=jax.ShapeDtypeStruct((16, NP, 32), jnp.float32),
        mesh=_MESH,
        compiler_params=_SC_PARAMS,
        scratch_types=[
            pltpu.VMEM_SHARED((NP, 32), jnp.float32),
            pltpu.VMEM((G, CHUNK), jnp.int32),
            pltpu.VMEM((G, CHUNK), jnp.int32),
            pltpu.VMEM((CHUNK, 32), jnp.float32),
            pltpu.SemaphoreType.DMA,
        ],
    )
    def _sums(tables, src3, dst3, z32, sums_out, acc, sstage, dstage, rows, sem):
        c = lax.axis_index("c")
        s = lax.axis_index("s")
        for e in range(8):
            k_in = 2 * e + c
            k_out = 2 * (SIG[e] if permute_out else e) + c
            pltpu.sync_copy(z32, acc.at[pl.ds(s * STRIPE, STRIPE)])
            plsc.subcore_barrier()
            tab = tables.at[k_in]

            def group(g, carry):
                pltpu.sync_copy(src3.at[e, s, g], sstage)
                pltpu.sync_copy(dst3.at[e, s, g], dstage)

                def chunk(j, carry2):
                    pltpu.async_copy(tab.at[sstage.at[j]], rows, sem).wait()
                    pltpu.sync_copy(rows, acc.at[dstage.at[j]], add=True)
                    return carry2

                lax.fori_loop(0, G, chunk, 0)
                return carry

            lax.fori_loop(0, NG, group, 0)
            plsc.subcore_barrier()
            pltpu.sync_copy(acc.at[pl.ds(s * STRIPE, STRIPE)],
                            sums_out.at[k_out, pl.ds(s * STRIPE, STRIPE)])
            plsc.subcore_barrier()

    return _sums


_sc_sums_l1 = _make_sc_sums(True)
_sc_sums_l2 = _make_sc_sums(False)


def _d1_body(s1_ref, c_ref, o_ref):
    inv = 1.0 / jnp.maximum(c_ref[...], 1.0)
    o_ref[...] = s1_ref[...] * inv


def _tc_divide(s1p, cntp):
    r1 = 3136
    nb = NP // r1
    return pl.pallas_call(
        _d1_body,
        grid=(16, nb),
        in_specs=[
            pl.BlockSpec((1, r1, 32), lambda k, b: (k, b, 0)),
            pl.BlockSpec((1, r1, 32), lambda k, b: (k // 2, b, 0)),
        ],
        out_specs=pl.BlockSpec((1, r1, 32), lambda k, b: (k, b, 0)),
        out_shape=jax.ShapeDtypeStruct((16, NP, 32), jnp.float32),
    )(s1p, cntp)


def _final_body(u_ref, v_ref, p_ref, t_ref, t2_ref, s2_ref, c_ref,
                ou_ref, ov_ref, op_ref, ot_ref):
    embs = [u_ref, v_ref, p_ref, t_ref]
    outs = [ou_ref, ov_ref, op_ref, ot_ref]
    third = 1.0 / 3.0
    for nt in range(4):
        for q in range(4):
            e_p = NT_ETYPES[nt][q // 2]
            cq = q % 2
            m1 = t2_ref[2 * SIG[e_p] + cq]
            inv = 1.0 / jnp.maximum(c_ref[SIG[e_p]], 1.0)
            m2 = s2_ref[2 * e_p + cq] * inv
            base = embs[nt][:, 32 * q:32 * (q + 1)]
            outs[nt][:, 32 * q:32 * (q + 1)] = base + 0.5 * m1 + third * m2


def _tc_final(u0, v0, p0, t0, t2_slab, s2_slab, cnt_slab):
    r = 1000
    nb = N // r
    emb_spec = pl.BlockSpec((r, 128), lambda b: (b, 0))
    return pl.pallas_call(
        _final_body,
        grid=(nb,),
        in_specs=[
            emb_spec, emb_spec, emb_spec, emb_spec,
            pl.BlockSpec((16, r, 32), lambda b: (0, b, 0)),
            pl.BlockSpec((16, r, 32), lambda b: (0, b, 0)),
            pl.BlockSpec((8, r, 32), lambda b: (0, b, 0)),
        ],
        out_specs=[emb_spec, emb_spec, emb_spec, emb_spec],
        out_shape=[jax.ShapeDtypeStruct((N, 128), jnp.float32)] * 4,
    )(u0, v0, p0, t0, t2_slab, s2_slab, cnt_slab)


def _build_tables(u, v, p, t):
    slabs = []
    for emb in (u, v, p, t):
        padded = jnp.pad(emb, ((0, NP - N), (0, 0)))
        slabs.append(padded.reshape(NP, 4, 32).transpose(1, 0, 2))
    return jnp.concatenate(slabs, axis=0)


def kernel(user_emb, video_emb, publisher_emb, tag_emb,
           edge_index_uv, edge_index_up, edge_index_vu, edge_index_vt,
           edge_index_pu, edge_index_pt, edge_index_tv, edge_index_tp):
    eis = [edge_index_uv, edge_index_up, edge_index_vu, edge_index_vt,
           edge_index_pu, edge_index_pt, edge_index_tv, edge_index_tp]
    srcs = jnp.stack([ei[0] for ei in eis])
    dsts = jnp.stack([ei[1] for ei in eis])
    src3 = jnp.pad(srcs, ((0, 0), (0, EP - E))).reshape(8, 16, NG, G, CHUNK)
    dst3 = jnp.pad(dsts, ((0, 0), (0, EP - E)),
                   constant_values=N).reshape(8, 16, NG, G, CHUNK)

    z32 = jnp.zeros((STRIPE, 32), jnp.float32)
    ones32 = jnp.ones((CHUNK, 32), jnp.float32)

    tables0 = _build_tables(user_emb, video_emb, publisher_emb, tag_emb)

    cntp = _sc_counts(dst3, z32, ones32)
    s1p = _sc_sums_l1(tables0, src3, dst3, z32)

    t2 = _tc_divide(s1p, cntp)

    s2 = _sc_sums_l2(t2, src3, dst3, z32)

    return _tc_final(user_emb, video_emb, publisher_emb, tag_emb, t2, s2, cntp)


# speedup vs baseline: 1.1783x; 1.1161x over previous
import functools

import jax
import jax.numpy as jnp
from jax import lax
from jax.experimental import pallas as pl
from jax.experimental.pallas import tpu as pltpu
from jax.experimental.pallas import tpu_sc as plsc

N = 50000
E = 500000
NP = 50176
STRIPE = NP // 16
CHUNK = 128
K_CHUNKS = 245
PT = CHUNK * K_CHUNKS
G = 49
EP = 16 * PT
NG = K_CHUNKS // G
NB = 1

ETYPES = ['uv', 'up', 'vu', 'vt', 'pu', 'pt', 'tv', 'tp']
SIG = [2, 4, 0, 6, 1, 7, 3, 5]
NT_ETYPES = [[2, 4], [0, 6], [1, 7], [3, 5]]

_MESH = plsc.VectorSubcoreMesh(core_axis_name="c", subcore_axis_name="s")
_SC_PARAMS = pltpu.CompilerParams(use_tc_tiling_on_sc=False)


@functools.partial(
    pl.kernel,
    out_type=jax.ShapeDtypeStruct((8, NP, 32), jnp.float32),
    mesh=_MESH,
    compiler_params=_SC_PARAMS,
    scratch_types=[
        pltpu.VMEM_SHARED((NP, 32), jnp.float32),
        pltpu.VMEM((G, CHUNK), jnp.int32),
        pltpu.VMEM((CHUNK, 32), jnp.float32),
    ],
)
def _sc_counts(dst3, z32, ones32, cnt_out, acc, stage, ones_v):
    c = lax.axis_index("c")
    s = lax.axis_index("s")
    pltpu.sync_copy(ones32, ones_v)
    for half in range(2):
        @pl.when(c == half)
        def _():
            for i in range(4):
                e = 4 * half + i
                pltpu.sync_copy(z32, acc.at[pl.ds(s * STRIPE, STRIPE)])
                plsc.subcore_barrier()

                def group(g, carry):
                    pltpu.sync_copy(dst3.at[e, s, g], stage)

                    def chunk(j, carry2):
                        pltpu.sync_copy(ones_v, acc.at[stage.at[j]], add=True)
                        return carry2

                    lax.fori_loop(0, G, chunk, 0)
                    return carry

                lax.fori_loop(0, NG, group, 0)
                plsc.subcore_barrier()
                pltpu.sync_copy(acc.at[pl.ds(s * STRIPE, STRIPE)],
                                cnt_out.at[SIG[e], pl.ds(s * STRIPE, STRIPE)])
                plsc.subcore_barrier()


def _make_sc_sums(permute_out):
    @functools.partial(
        pl.kernel,
        out_type=jax.ShapeDtypeStruct((16, NP, 32), jnp.float32),
        mesh=_MESH,
        compiler_params=_SC_PARAMS,
        scratch_types=[
            pltpu.VMEM_SHARED((NP, 32), jnp.float32),
            pltpu.VMEM((G, CHUNK), jnp.int32),
            pltpu.VMEM((G, CHUNK), jnp.int32),
            pltpu.VMEM((CHUNK, 32), jnp.float32),
            pltpu.SemaphoreType.DMA,
        ],
    )
    def _sums(tables, src3, dst3, z32, sums_out, acc, sstage, dstage, rows, sem):
        c = lax.axis_index("c")
        s = lax.axis_index("s")
        for e in range(8):
            k_in = 2 * e + c
            k_out = 2 * (SIG[e] if permute_out else e) + c
            pltpu.sync_copy(z32, acc.at[pl.ds(s * STRIPE, STRIPE)])
            plsc.subcore_barrier()
            tab = tables.at[k_in]

            def group(g, carry):
                pltpu.sync_copy(src3.at[e, s, g], sstage)
                pltpu.sync_copy(dst3.at[e, s, g], dstage)

                def chunk(j, carry2):
                    pltpu.async_copy(tab.at[sstage.at[j]], rows, sem).wait()
                    pltpu.sync_copy(rows, acc.at[dstage.at[j]], add=True)
                    return carry2

                lax.fori_loop(0, G, chunk, 0)
                return carry

            lax.fori_loop(0, NG, group, 0)
            plsc.subcore_barrier()
            pltpu.sync_copy(acc.at[pl.ds(s * STRIPE, STRIPE)],
                            sums_out.at[k_out, pl.ds(s * STRIPE, STRIPE)])
            plsc.subcore_barrier()

    return _sums


_sc_sums_l1 = _make_sc_sums(True)
_sc_sums_l2 = _make_sc_sums(False)


def _d1_body(s1_ref, c_ref, o_ref):
    inv = 1.0 / jnp.maximum(c_ref[...], 1.0)
    o_ref[...] = s1_ref[...] * inv


def _tc_divide(s1pr, cntpr):
    rows = NP // 4
    r1 = 1568
    nb = rows // r1
    return pl.pallas_call(
        _d1_body,
        grid=(16, nb),
        in_specs=[
            pl.BlockSpec((1, r1, 128), lambda k, b: (k, b, 0)),
            pl.BlockSpec((1, r1, 128), lambda k, b: (k // 2, b, 0)),
        ],
        out_specs=pl.BlockSpec((1, r1, 128), lambda k, b: (k, b, 0)),
        out_shape=jax.ShapeDtypeStruct((16, rows, 128), jnp.float32),
    )(s1pr, cntpr)


def _final_body(u_ref, v_ref, p_ref, t_ref, t2_ref, s2_ref, c_ref,
                ou_ref, ov_ref, op_ref, ot_ref):
    embs = [u_ref, v_ref, p_ref, t_ref]
    outs = [ou_ref, ov_ref, op_ref, ot_ref]
    third = 1.0 / 3.0
    for nt in range(4):
        for q in range(4):
            e_p = NT_ETYPES[nt][q // 2]
            cq = q % 2
            m1 = t2_ref[2 * SIG[e_p] + cq]
            inv = 1.0 / jnp.maximum(c_ref[SIG[e_p]], 1.0)
            m2 = s2_ref[2 * e_p + cq] * inv
            base = embs[nt][:, 32 * q:32 * (q + 1)]
            outs[nt][:, 32 * q:32 * (q + 1)] = base + 0.5 * m1 + third * m2


def _tc_final(u0, v0, p0, t0, t2_slab, s2_slab, cnt_slab):
    r = 1000
    nb = N // r
    emb_spec = pl.BlockSpec((r, 128), lambda b: (b, 0))
    return pl.pallas_call(
        _final_body,
        grid=(nb,),
        in_specs=[
            emb_spec, emb_spec, emb_spec, emb_spec,
            pl.BlockSpec((16, r, 32), lambda b: (0, b, 0)),
            pl.BlockSpec((16, r, 32), lambda b: (0, b, 0)),
            pl.BlockSpec((8, r, 32), lambda b: (0, b, 0)),
        ],
        out_specs=[emb_spec, emb_spec, emb_spec, emb_spec],
        out_shape=[jax.ShapeDtypeStruct((N, 128), jnp.float32)] * 4,
    )(u0, v0, p0, t0, t2_slab, s2_slab, cnt_slab)


def _build_tables(u, v, p, t):
    slabs = []
    for emb in (u, v, p, t):
        padded = jnp.pad(emb, ((0, NP - N), (0, 0)))
        slabs.append(padded.reshape(NP, 4, 32).transpose(1, 0, 2))
    return jnp.concatenate(slabs, axis=0)


def kernel(user_emb, video_emb, publisher_emb, tag_emb,
           edge_index_uv, edge_index_up, edge_index_vu, edge_index_vt,
           edge_index_pu, edge_index_pt, edge_index_tv, edge_index_tp):
    eis = [edge_index_uv, edge_index_up, edge_index_vu, edge_index_vt,
           edge_index_pu, edge_index_pt, edge_index_tv, edge_index_tp]
    srcs = jnp.stack([ei[0] for ei in eis])
    dsts = jnp.stack([ei[1] for ei in eis])
    src3 = jnp.pad(srcs, ((0, 0), (0, EP - E))).reshape(8, 16, NG, G, CHUNK)
    dst3 = jnp.pad(dsts, ((0, 0), (0, EP - E)),
                   constant_values=N).reshape(8, 16, NG, G, CHUNK)

    z32 = jnp.zeros((STRIPE, 32), jnp.float32)
    ones32 = jnp.ones((CHUNK, 32), jnp.float32)

    tables0 = _build_tables(user_emb, video_emb, publisher_emb, tag_emb)

    cntp = _sc_counts(dst3, z32, ones32)
    s1p = _sc_sums_l1(tables0, src3, dst3, z32)

    t2r = _tc_divide(s1p.reshape(16, NP // 4, 128),
                     cntp.reshape(8, NP // 4, 128))
    t2 = t2r.reshape(16, NP, 32)

    s2 = _sc_sums_l2(t2, src3, dst3, z32)

    return _tc_final(user_emb, video_emb, publisher_emb, tag_emb, t2, s2, cntp)
